# Initial kernel scaffold; baseline (speedup 1.0000x reference)
import jax
import jax.numpy as jnp
from jax.experimental import pallas as pl


def kernel(output, target):
    raise NotImplementedError("write your pallas kernel here")


# speedup vs baseline: 8.3672x; 8.3672x over previous
import numpy as np

import jax
import jax.numpy as jnp
from jax import lax
from jax.experimental import pallas as pl
from jax.experimental.pallas import tpu as pltpu
from jax.experimental.pallas import tpu_sc as plsc

GRID_NUM = 7.0
LAMBDA_COORD = 5.0
LAMBDA_NOOBJ = 0.5

BATCH = 2048
CELLS = 7 * 7
COLS = 26
N_ROWS = BATCH * CELLS
NC, NS = 2, 16
NW = NC * NS
ROWS_PER_W = N_ROWS // NW
CHUNK_ROWS = 784
N_CHUNKS = ROWS_PER_W // CHUNK_ROWS
GROUPS = CHUNK_ROWS // 16
CHUNK_W = CHUNK_ROWS * COLS

_RSQRT_MAGIC = np.int32(0x5F3759DF)


def _sqrt16(x):
    xc = jnp.maximum(x, jnp.float32(1e-12))
    i = lax.bitcast_convert_type(xc, jnp.int32)
    i = _RSQRT_MAGIC - lax.shift_right_arithmetic(i, 1)
    y = lax.bitcast_convert_type(i, jnp.float32)
    half = jnp.float32(0.5) * xc
    for _ in range(3):
        y = y * (jnp.float32(1.5) - half * y * y)
    return xc * y


def _corners(cx, cy, w, h):
    gx = cx * jnp.float32(1.0 / GRID_NUM)
    gy = cy * jnp.float32(1.0 / GRID_NUM)
    hw = jnp.float32(0.5) * w
    hh = jnp.float32(0.5) * h
    return gx - hw, gy - hh, gx + hw, gy + hh


def _iou16(p, t):
    px0, py0, px1, py1 = p
    tx0, ty0, tx1, ty1 = t
    ltx = jnp.maximum(px0, tx0)
    lty = jnp.maximum(py0, ty0)
    rbx = jnp.minimum(px1, tx1)
    rby = jnp.minimum(py1, ty1)
    iw = jnp.maximum(rbx - ltx, jnp.float32(0.0))
    ih = jnp.maximum(rby - lty, jnp.float32(0.0))
    inter = iw * ih
    ap = (px1 - px0) * (py1 - py0)
    at = (tx1 - tx0) * (ty1 - ty0)
    return inter / (ap + at - inter)


def _loss_body(o_hbm, t_hbm, out_hbm, ob, tb, vstage):
    c = lax.axis_index("c")
    s = lax.axis_index("s")
    wid = s * NC + c
    lane = lax.iota(jnp.int32, 16)
    col0 = lane * COLS

    def group_body(g, acc):
        base = col0 + g * (16 * COLS)

        def go(col):
            return plsc.load_gather(ob, [base + col])

        def gt(col):
            return plsc.load_gather(tb, [base + col])

        cls = jnp.zeros((16,), jnp.float32)
        for j in range(10, 26):
            d = go(j) - gt(j)
            cls = cls + d * d

        po = [go(j) for j in range(10)]
        to = [gt(j) for j in range(10)]

        p1 = _corners(po[0], po[1], po[2], po[3])
        p2 = _corners(po[5], po[6], po[7], po[8])
        t1 = _corners(to[0], to[1], to[2], to[3])
        t2 = _corners(to[5], to[6], to[7], to[8])
        iou1 = _iou16(p1, t1)
        iou2 = _iou16(p2, t2)
        sel2 = iou2 > iou1
        iou_max = jnp.maximum(iou1, iou2)

        pcx = jnp.where(sel2, po[5], po[0])
        pcy = jnp.where(sel2, po[6], po[1])
        pw = jnp.where(sel2, po[7], po[2])
        ph = jnp.where(sel2, po[8], po[3])
        pcf = jnp.where(sel2, po[9], po[4])
        pnr = jnp.where(sel2, po[4], po[9])
        tcx = jnp.where(sel2, to[5], to[0])
        tcy = jnp.where(sel2, to[6], to[1])
        tw = jnp.where(sel2, to[7], to[2])
        th = jnp.where(sel2, to[8], to[3])

        dconf = pcf - iou_max
        dx = pcx - tcx
        dy = pcy - tcy
        xy = dx * dx + dy * dy
        wh = (pw + tw - jnp.float32(2.0) * _sqrt16(pw * tw)
              + ph + th - jnp.float32(2.0) * _sqrt16(ph * th))

        row = (dconf * dconf
               + jnp.float32(LAMBDA_COORD) * (xy + wh)
               + jnp.float32(LAMBDA_NOOBJ) * pnr * pnr
               + cls)
        objf = jnp.where(gt(4) > jnp.float32(0.0),
                         jnp.float32(1.0), jnp.float32(0.0))
        return acc + objf * row

    acc = jnp.zeros((16,), jnp.float32)
    for chunk in range(N_CHUNKS):
        base = (wid * ROWS_PER_W + chunk * CHUNK_ROWS) * COLS
        pltpu.sync_copy(o_hbm.at[pl.ds(base, CHUNK_W)], ob)
        pltpu.sync_copy(t_hbm.at[pl.ds(base, CHUNK_W)], tb)
        acc = lax.fori_loop(0, GROUPS, group_body, acc)

    vstage[...] = acc
    pltpu.sync_copy(vstage, out_hbm.at[wid])


@jax.jit
def _sc_partials(o_flat, t_flat):
    mesh = plsc.VectorSubcoreMesh(
        core_axis_name="c", subcore_axis_name="s",
        num_cores=NC, num_subcores=NS)
    return pl.kernel(
        _loss_body,
        out_type=jax.ShapeDtypeStruct((NW, 16), jnp.float32),
        mesh=mesh,
        scratch_types=[
            pltpu.VMEM((CHUNK_W,), jnp.float32),
            pltpu.VMEM((CHUNK_W,), jnp.float32),
            pltpu.VMEM((16,), jnp.float32),
        ],
        compiler_params=pltpu.CompilerParams(needs_layout_passes=False),
    )(o_flat, t_flat)


def kernel(output, target):
    part = _sc_partials(output.reshape(-1), target.reshape(-1))
    return jnp.sum(part) / jnp.float32(BATCH)


# speedup vs baseline: 8.8802x; 1.0613x over previous
import numpy as np

import jax
import jax.numpy as jnp
from jax import lax
from jax.experimental import pallas as pl
from jax.experimental.pallas import tpu as pltpu
from jax.experimental.pallas import tpu_sc as plsc

GRID_NUM = 7.0
LAMBDA_COORD = 5.0
LAMBDA_NOOBJ = 0.5

BATCH = 2048
CELLS = 7 * 7
COLS = 26
N_ROWS = BATCH * CELLS
NC, NS = 2, 16
NW = NC * NS
ROWS_PER_W = N_ROWS // NW
CHUNK_ROWS = 784
N_CHUNKS = ROWS_PER_W // CHUNK_ROWS
GROUPS = CHUNK_ROWS // 16
CHUNK_W = CHUNK_ROWS * COLS

_RSQRT_MAGIC = np.int32(0x5F3759DF)


def _sqrt16(x):
    xc = jnp.maximum(x, jnp.float32(1e-12))
    i = lax.bitcast_convert_type(xc, jnp.int32)
    i = _RSQRT_MAGIC - lax.shift_right_arithmetic(i, 1)
    y = lax.bitcast_convert_type(i, jnp.float32)
    half = jnp.float32(0.5) * xc
    for _ in range(3):
        y = y * (jnp.float32(1.5) - half * y * y)
    return xc * y


def _corners(cx, cy, w, h):
    gx = cx * jnp.float32(1.0 / GRID_NUM)
    gy = cy * jnp.float32(1.0 / GRID_NUM)
    hw = jnp.float32(0.5) * w
    hh = jnp.float32(0.5) * h
    return gx - hw, gy - hh, gx + hw, gy + hh


def _iou16(p, t):
    px0, py0, px1, py1 = p
    tx0, ty0, tx1, ty1 = t
    ltx = jnp.maximum(px0, tx0)
    lty = jnp.maximum(py0, ty0)
    rbx = jnp.minimum(px1, tx1)
    rby = jnp.minimum(py1, ty1)
    iw = jnp.maximum(rbx - ltx, jnp.float32(0.0))
    ih = jnp.maximum(rby - lty, jnp.float32(0.0))
    inter = iw * ih
    ap = (px1 - px0) * (py1 - py0)
    at = (tx1 - tx0) * (ty1 - ty0)
    return inter / (ap + at - inter)


def _loss_body(o_hbm, t_hbm, out_hbm, ob0, tb0, ob1, tb1, vstage,
               sem0, sem1):
    c = lax.axis_index("c")
    s = lax.axis_index("s")
    wid = s * NC + c
    lane = lax.iota(jnp.int32, 16)
    col0 = lane * COLS
    bufs = ((ob0, tb0), (ob1, tb1))
    sems = (sem0, sem1)

    def make_group_body(ob, tb):
      def group_body(g, acc):
        base = col0 + g * (16 * COLS)

        def go(col):
            return plsc.load_gather(ob, [base + col])

        def gt(col):
            return plsc.load_gather(tb, [base + col])

        cls = jnp.zeros((16,), jnp.float32)
        for j in range(10, 26):
            d = go(j) - gt(j)
            cls = cls + d * d

        po = [go(j) for j in range(10)]
        to = [gt(j) for j in range(10)]

        p1 = _corners(po[0], po[1], po[2], po[3])
        p2 = _corners(po[5], po[6], po[7], po[8])
        t1 = _corners(to[0], to[1], to[2], to[3])
        t2 = _corners(to[5], to[6], to[7], to[8])
        iou1 = _iou16(p1, t1)
        iou2 = _iou16(p2, t2)
        sel2 = iou2 > iou1
        iou_max = jnp.maximum(iou1, iou2)

        pcx = jnp.where(sel2, po[5], po[0])
        pcy = jnp.where(sel2, po[6], po[1])
        pw = jnp.where(sel2, po[7], po[2])
        ph = jnp.where(sel2, po[8], po[3])
        pcf = jnp.where(sel2, po[9], po[4])
        pnr = jnp.where(sel2, po[4], po[9])
        tcx = jnp.where(sel2, to[5], to[0])
        tcy = jnp.where(sel2, to[6], to[1])
        tw = jnp.where(sel2, to[7], to[2])
        th = jnp.where(sel2, to[8], to[3])

        dconf = pcf - iou_max
        dx = pcx - tcx
        dy = pcy - tcy
        xy = dx * dx + dy * dy
        wh = (pw + tw - jnp.float32(2.0) * _sqrt16(pw * tw)
              + ph + th - jnp.float32(2.0) * _sqrt16(ph * th))

        row = (dconf * dconf
               + jnp.float32(LAMBDA_COORD) * (xy + wh)
               + jnp.float32(LAMBDA_NOOBJ) * pnr * pnr
               + cls)
        objf = jnp.where(to[4] > jnp.float32(0.0),
                         jnp.float32(1.0), jnp.float32(0.0))
        return acc + objf * row

      return group_body

    pend = {}

    def start(chunk, slot):
        base = (wid * ROWS_PER_W + chunk * CHUNK_ROWS) * COLS
        pend[slot] = (
            pltpu.async_copy(o_hbm.at[pl.ds(base, CHUNK_W)],
                             bufs[slot][0], sems[slot]),
            pltpu.async_copy(t_hbm.at[pl.ds(base, CHUNK_W)],
                             bufs[slot][1], sems[slot]),
        )

    acc = jnp.zeros((16,), jnp.float32)
    start(0, 0)
    for chunk in range(N_CHUNKS):
        slot = chunk % 2
        if chunk + 1 < N_CHUNKS:
            start(chunk + 1, (chunk + 1) % 2)
        for cp in pend[slot]:
            cp.wait()
        acc = plsc.parallel_loop(0, GROUPS, 1, unroll=2, carry=acc)(
            make_group_body(*bufs[slot]))

    vstage[...] = acc
    pltpu.sync_copy(vstage, out_hbm.at[wid])


@jax.jit
def _sc_partials(o_flat, t_flat):
    mesh = plsc.VectorSubcoreMesh(
        core_axis_name="c", subcore_axis_name="s",
        num_cores=NC, num_subcores=NS)
    return pl.kernel(
        _loss_body,
        out_type=jax.ShapeDtypeStruct((NW, 16), jnp.float32),
        mesh=mesh,
        scratch_types=[
            pltpu.VMEM((CHUNK_W,), jnp.float32),
            pltpu.VMEM((CHUNK_W,), jnp.float32),
            pltpu.VMEM((CHUNK_W,), jnp.float32),
            pltpu.VMEM((CHUNK_W,), jnp.float32),
            pltpu.VMEM((16,), jnp.float32),
            pltpu.SemaphoreType.DMA,
            pltpu.SemaphoreType.DMA,
        ],
        compiler_params=pltpu.CompilerParams(needs_layout_passes=False),
    )(o_flat, t_flat)


def kernel(output, target):
    part = _sc_partials(output.reshape(-1), target.reshape(-1))
    return jnp.sum(part) / jnp.float32(BATCH)
